# baseline (speedup 1.0000x reference)
import functools

import jax
import jax.numpy as jnp
from jax import lax
from jax.experimental import pallas as pl
from jax.experimental.pallas import tpu as pltpu
from jax.experimental.pallas import tpu_sc as plsc

DIM = 128
DE = 16
HID = 128
TW = 256
NC = 2
NS = 16
NW = NC * NS
LANES = 16
EB = 128
GRP = 8
BE = 2048
ALIGN = NW * EB * GRP


def _pack_pairs(p):
    u32 = jnp.uint32
    be = jax.lax.bitcast_convert_type(p[:, :64], u32) + u32(0x8000)
    bo = jax.lax.bitcast_convert_type(p[:, 64:], u32) + u32(0x8000)
    pk = (be & u32(0xFFFF0000)) | (bo >> 16)
    return jax.lax.bitcast_convert_type(pk, jnp.float32)


def _pre_body(h_ref, cp_ref, ws_ref, wd_ref, ts_ref, td_ref):
    hb = h_ref[...]
    cp = cp_ref[...]
    bn = hb.shape[0]
    z = jnp.zeros((bn, 48), jnp.float32)
    ts_ref[...] = jnp.concatenate(
        [_pack_pairs(jnp.dot(hb, ws_ref[...])), cp, z], axis=1)
    td_ref[...] = jnp.concatenate(
        [_pack_pairs(jnp.dot(hb, wd_ref[...])), -cp, z], axis=1)


@functools.lru_cache(maxsize=None)
def _pre_call(n):
    bn = 1000 if n % 1000 == 0 else n
    grid = n // bn
    f32 = jnp.float32
    return pl.pallas_call(
        _pre_body,
        grid=(grid,),
        in_specs=[
            pl.BlockSpec((bn, DIM), lambda i: (i, 0)),
            pl.BlockSpec((bn, 16), lambda i: (i, 0)),
            pl.BlockSpec((DIM, DIM), lambda i: (0, 0)),
            pl.BlockSpec((DIM, DIM), lambda i: (0, 0)),
        ],
        out_specs=[
            pl.BlockSpec((bn, DIM), lambda i: (i, 0)),
            pl.BlockSpec((bn, DIM), lambda i: (i, 0)),
        ],
        out_shape=[
            jax.ShapeDtypeStruct((n, DIM), f32),
            jax.ShapeDtypeStruct((n, DIM), f32),
        ],
    )


def _edge_body(e_real, off, gs_ref, gd_ref, ea_ref, we2_ref, wc1_ref, wrad_ref,
               wea_ref, wc2_ref, be1_ref, be2_ref, bc1_ref, f_ref, s_ref):
    f32 = jnp.float32
    u32 = jnp.uint32
    gs = gs_ref[...]
    gd = gd_ref[...]

    def unpack(x):
        b = jax.lax.bitcast_convert_type(x, u32)
        hi = jax.lax.bitcast_convert_type(b & u32(0xFFFF0000), f32)
        lo = jax.lax.bitcast_convert_type(b << 16, f32)
        return hi, lo

    shi, slo = unpack(gs[:, :64])
    dhi, dlo = unpack(gd[:, :64])
    ssum = jnp.concatenate([shi + dhi, slo + dlo], axis=1)
    cd = gs[:, 64:80] + gd[:, 64:80]
    radial = jnp.sum(cd * cd, axis=1, keepdims=True)
    pre = (ssum + radial * wrad_ref[...] + jnp.dot(ea_ref[...], wea_ref[...])
           + be1_ref[...])
    f1 = pre * jax.nn.sigmoid(pre)
    h2 = jnp.dot(f1, we2_ref[...]) + be2_ref[...]
    f2 = h2 * jax.nn.sigmoid(h2)
    h3 = jnp.dot(f2, wc1_ref[...]) + bc1_ref[...]
    f3 = h3 * jax.nn.sigmoid(h3)
    t = jnp.sum(f3 * wc2_ref[...], axis=1, keepdims=True)
    eid = (off + pl.program_id(0) * BE
           + lax.broadcasted_iota(jnp.int32, (BE, 1), 0))
    valid = eid < e_real
    f_ref[...] = jnp.where(valid, f2, 0.0)
    sm = cd * t
    lane = lax.broadcasted_iota(jnp.int32, (BE, 16), 1)
    sm = jnp.where(lane == 3, 1.0, sm)
    s_ref[...] = jnp.where(valid, sm, 0.0)


@functools.lru_cache(maxsize=None)
def _edge_call(ep, e_real, off):
    f32 = jnp.float32
    full = lambda i: (0, 0)
    return pl.pallas_call(
        functools.partial(_edge_body, e_real, off),
        grid=(ep // BE,),
        in_specs=[
            pl.BlockSpec((BE, DIM), lambda i: (i, 0)),
            pl.BlockSpec((BE, DIM), lambda i: (i, 0)),
            pl.BlockSpec((BE, DE), lambda i: (i, 0)),
            pl.BlockSpec((HID, HID), full),
            pl.BlockSpec((HID, HID), full),
            pl.BlockSpec((1, HID), full),
            pl.BlockSpec((DE, HID), full),
            pl.BlockSpec((1, HID), full),
            pl.BlockSpec((1, HID), full),
            pl.BlockSpec((1, HID), full),
            pl.BlockSpec((1, HID), full),
        ],
        out_specs=[
            pl.BlockSpec((BE, DIM), lambda i: (i, 0)),
            pl.BlockSpec((BE, 16), lambda i: (i, 0)),
        ],
        out_shape=[
            jax.ShapeDtypeStruct((ep, DIM), f32),
            jax.ShapeDtypeStruct((ep, 16), f32),
        ],
    )


def _node_body(h_ref, cp_ref, pf_ref, ps_ref, pf2_ref, ps2_ref, wn1h_ref,
               wn1a_ref, wn2_ref, bn1_ref, bn2_ref, ho_ref, co_ref):
    hb = h_ref[...]
    pf = pf_ref[...]
    pf2 = pf2_ref[...]
    aggf = pf[0] + pf[1] + pf2[0] + pf2[1]
    ps = ps_ref[...]
    ps2 = ps2_ref[...]
    aggs = (ps[0, :, :16] + ps[1, :, :16]
            + ps2[0, :, :16] + ps2[1, :, :16])
    cnt = jnp.maximum(aggs[:, 3:4], 1.0)
    lane = lax.broadcasted_iota(jnp.int32, aggs.shape, 1)
    tr = jnp.where(lane < 3, aggs, 0.0)
    co_ref[...] = cp_ref[...] + tr / cnt
    u = (jnp.dot(hb, wn1h_ref[...]) + jnp.dot(aggf, wn1a_ref[...])
         + bn1_ref[...])
    su = u * jax.nn.sigmoid(u)
    ho_ref[...] = hb + jnp.dot(su, wn2_ref[...]) + bn2_ref[...]


@functools.lru_cache(maxsize=None)
def _node_call(n):
    bn = 1000 if n % 1000 == 0 else n
    grid = n // bn
    f32 = jnp.float32
    full = lambda i: (0, 0)
    return pl.pallas_call(
        _node_body,
        grid=(grid,),
        in_specs=[
            pl.BlockSpec((bn, DIM), lambda i: (i, 0)),
            pl.BlockSpec((bn, 16), lambda i: (i, 0)),
            pl.BlockSpec((2, bn, DIM), lambda i: (0, i, 0)),
            pl.BlockSpec((2, bn, DIM), lambda i: (0, i, 0)),
            pl.BlockSpec((2, bn, DIM), lambda i: (0, i, 0)),
            pl.BlockSpec((2, bn, DIM), lambda i: (0, i, 0)),
            pl.BlockSpec((DIM, HID), full),
            pl.BlockSpec((DIM, HID), full),
            pl.BlockSpec((HID, DIM), full),
            pl.BlockSpec((1, HID), full),
            pl.BlockSpec((1, DIM), full),
        ],
        out_specs=[
            pl.BlockSpec((bn, DIM), lambda i: (i, 0)),
            pl.BlockSpec((bn, 16), lambda i: (i, 0)),
        ],
        out_shape=[
            jax.ShapeDtypeStruct((n, DIM), f32),
            jax.ShapeDtypeStruct((n, 16), f32),
        ],
    )


@functools.lru_cache(maxsize=None)
def _gather_call(ep):
    f32 = jnp.float32
    nb = ep // (NW * EB)
    ng = nb // GRP
    mesh = plsc.VectorSubcoreMesh(core_axis_name="c", subcore_axis_name="s")

    @functools.partial(
        pl.kernel,
        mesh=mesh,
        out_type=[
            jax.ShapeDtypeStruct((ep, DIM), f32),
            jax.ShapeDtypeStruct((ep, DIM), f32),
        ],
        scratch_types=[
            pltpu.VMEM((GRP, EB), jnp.int32),
            pltpu.VMEM((GRP, EB), jnp.int32),
            pltpu.VMEM((EB, DIM), f32),
            pltpu.VMEM((EB, DIM), f32),
            pltpu.VMEM((EB, DIM), f32),
            pltpu.VMEM((EB, DIM), f32),
            pltpu.SemaphoreType.DMA,
            pltpu.SemaphoreType.DMA,
            pltpu.SemaphoreType.DMA,
            pltpu.SemaphoreType.DMA,
        ],
    )
    def gather_k(tsrc, tdst, rows, cols, gs_out, gd_out, rowv, colv,
                 gs0, gs1, gd0, gd1, gsem0, gsem1, wsem0, wsem1):
        wid = lax.axis_index("s") * NC + lax.axis_index("c")
        base_row = wid * nb
        gsb = (gs0, gs1)
        gdb = (gd0, gd1)
        gsem = (gsem0, gsem1)
        wsem = (wsem0, wsem1)

        def body(g, carry):
            r0 = pl.multiple_of(base_row + g * GRP, GRP)
            pltpu.sync_copy(rows.at[pl.ds(r0, GRP)], rowv)
            pltpu.sync_copy(cols.at[pl.ds(r0, GRP)], colv)
            gh = [None, None]
            wh = [None, None]
            gh[0] = (pltpu.async_copy(tsrc.at[rowv.at[0]], gsb[0], gsem[0]),
                     pltpu.async_copy(tdst.at[colv.at[0]], gdb[0], gsem[0]))
            for j in range(GRP):
                b = j & 1
                nx = 1 - b
                for c in gh[b]:
                    c.wait()
                if j < GRP - 1:
                    if wh[nx] is not None:
                        for w in wh[nx]:
                            w.wait()
                    gh[nx] = (
                        pltpu.async_copy(tsrc.at[rowv.at[j + 1]], gsb[nx],
                                         gsem[nx]),
                        pltpu.async_copy(tdst.at[colv.at[j + 1]], gdb[nx],
                                         gsem[nx]))
                e0 = pl.multiple_of((r0 + j) * EB, EB)
                wh[b] = (
                    pltpu.async_copy(gsb[b], gs_out.at[pl.ds(e0, EB)],
                                     wsem[b]),
                    pltpu.async_copy(gdb[b], gd_out.at[pl.ds(e0, EB)],
                                     wsem[b]))
            for pair in wh:
                for w in pair:
                    w.wait()
            return carry

        lax.fori_loop(0, ng, body, 0)

    return gather_k


@functools.lru_cache(maxsize=None)
def _scatter_call(ep, n):
    f32 = jnp.float32
    nb = ep // (NW * EB)
    ng = nb // GRP
    cs = EB
    mesh = plsc.VectorSubcoreMesh(core_axis_name="c", subcore_axis_name="s")

    @functools.partial(
        pl.kernel,
        mesh=mesh,
        out_type=[
            jax.ShapeDtypeStruct((NC, n, DIM), f32),
            jax.ShapeDtypeStruct((NC, n, DIM), f32),
        ],
        scratch_types=[
            pltpu.VMEM((GRP, EB), jnp.int32),
            pltpu.VMEM((cs, DIM), f32),
            pltpu.VMEM((16, DIM), f32),
            pltpu.VMEM((cs, DIM), f32),
            pltpu.VMEM_SHARED((n, DIM), f32),
        ],
    )
    def scatter_k(feat, small, rows, pf_out, ps_out, idxv, bf, bs16, bex, af):
        cid = lax.axis_index("c")
        sid = lax.axis_index("s")
        wid = sid * NC + cid
        stride = (n // (NS * 8)) * 8
        swid = -(-(n - (NS - 1) * stride) // cs) * cs
        base_n = pl.multiple_of(sid * stride, 8)
        base_row = wid * nb

        def zero_buf(buf):
            def zf(i, carry):
                buf[i // 8, pl.ds((i % 8) * LANES, LANES)] = jnp.zeros(
                    (LANES,), f32)
                return carry
            lax.fori_loop(0, cs * (DIM // LANES), zf, 0)

        def zero_stripe():
            for off in range(0, swid, cs):
                b = pl.multiple_of(base_n + off, 8)
                pltpu.sync_copy(bf, af.at[pl.ds(b, cs)])

        def add_pass(payload):
            def body(g, carry):
                r0 = pl.multiple_of(base_row + g * GRP, GRP)
                pltpu.sync_copy(rows.at[pl.ds(r0, GRP)], idxv)
                for j in range(GRP):
                    e0 = pl.multiple_of((r0 + j) * EB, EB)
                    pltpu.sync_copy(payload.at[pl.ds(e0, cs)], bf)
                    pltpu.sync_copy(bf, af.at[idxv.at[j]], add=True)
                return carry
            lax.fori_loop(0, ng, body, 0)

        def small_pass(spacked):
            def body(g, carry):
                r0 = pl.multiple_of(base_row + g * GRP, GRP)
                pltpu.sync_copy(rows.at[pl.ds(r0, GRP)], idxv)
                for j in range(GRP):
                    p0 = pl.multiple_of((r0 + j) * 16, 16)
                    pltpu.sync_copy(spacked.at[pl.ds(p0, 16)], bs16)
                    for r in range(16):
                        for k in range(8):
                            bex[8 * r + k, pl.ds(0, LANES)] = (
                                bs16[r, pl.ds(k * LANES, LANES)])
                    pltpu.sync_copy(bex, af.at[idxv.at[j]], add=True)
                return carry
            lax.fori_loop(0, ng, body, 0)

        def drain(out):
            for off in range(0, swid, cs):
                b = pl.multiple_of(base_n + off, 8)
                pltpu.sync_copy(af.at[pl.ds(b, cs)], bf)
                pltpu.sync_copy(bf, out.at[cid, pl.ds(b, cs)])

        zero_buf(bf)
        zero_buf(bex)
        zero_stripe()
        plsc.subcore_barrier()
        add_pass(feat)
        plsc.subcore_barrier()
        drain(pf_out)
        plsc.subcore_barrier()
        zero_buf(bf)
        zero_stripe()
        plsc.subcore_barrier()
        small_pass(small)
        plsc.subcore_barrier()
        drain(ps_out)

    return scatter_k


def kernel(h, edge_index, coord, edge_attr, We1, be1, We2, be2, Wn1, bn1,
           Wn2, bn2, Wc1, bc1, Wc2):
    f32 = jnp.float32
    n = h.shape[0]
    e_real = edge_index.shape[1]
    ep = -(-e_real // ALIGN) * ALIGN
    pad = ep - e_real

    row = edge_index[0].astype(jnp.int32)
    col = edge_index[1].astype(jnp.int32)
    rowp = jnp.concatenate([row, jnp.zeros((pad,), jnp.int32)]).reshape(
        ep // EB, EB)
    colp = jnp.concatenate([col, jnp.zeros((pad,), jnp.int32)]).reshape(
        ep // EB, EB)
    eap = jnp.concatenate([edge_attr.astype(f32),
                           jnp.zeros((pad, DE), f32)], axis=0)
    cp = jnp.concatenate([coord.astype(f32), jnp.zeros((n, 13), f32)], axis=1)

    perm = jnp.arange(HID).reshape(HID // 2, 2).T.reshape(HID)
    Wsrc = We1[:DIM, perm]
    Wdst = We1[DIM:2 * DIM, perm]
    wrad = We1[2 * DIM:2 * DIM + 1, perm]
    Wea = We1[2 * DIM + 1:, perm]
    We2p = We2[perm, :]
    wc2r = Wc2.reshape(1, HID)
    Wn1h = Wn1[:DIM]
    Wn1a = Wn1[DIM:]
    be1r = be1[perm].reshape(1, HID)
    be2r = be2.reshape(1, HID)
    bc1r = bc1.reshape(1, HID)
    bn1r = bn1.reshape(1, HID)
    bn2r = bn2.reshape(1, DIM)

    ts, td = _pre_call(n)(h, cp, Wsrc, Wdst)
    half = ep // 2
    hbr = half // EB
    g1s, g1d = _gather_call(half)(ts, td, rowp[:hbr], colp[:hbr])
    g2s, g2d = _gather_call(half)(ts, td, rowp[hbr:], colp[hbr:])
    f1, s1 = _edge_call(half, e_real, 0)(g1s, g1d, eap[:half], We2p, Wc1,
                                         wrad, Wea, wc2r, be1r, be2r, bc1r)
    f2, s2 = _edge_call(half, e_real, half)(g2s, g2d, eap[half:], We2p, Wc1,
                                            wrad, Wea, wc2r, be1r, be2r, bc1r)
    pf1, ps1 = _scatter_call(half, n)(f1, s1.reshape(half // 8, 128),
                                      rowp[:hbr])
    pf2, ps2 = _scatter_call(half, n)(f2, s2.reshape(half // 8, 128),
                                      rowp[hbr:])
    ho, co = _node_call(n)(h, cp, pf1, ps1, pf2, ps2, Wn1h, Wn1a, Wn2,
                           bn1r, bn2r)
    return (ho, co[:, :3], edge_attr)


# speedup vs baseline: 1.0216x; 1.0216x over previous
import functools

import jax
import jax.numpy as jnp
from jax import lax
from jax.experimental import pallas as pl
from jax.experimental.pallas import tpu as pltpu
from jax.experimental.pallas import tpu_sc as plsc

DIM = 128
DE = 16
HID = 128
TW = 256
NC = 2
NS = 16
NW = NC * NS
LANES = 16
EB = 128
GRP = 8
BE = 2048
ALIGN = NW * EB * GRP


def _pack_pairs(p):
    u32 = jnp.uint32
    be = jax.lax.bitcast_convert_type(p[:, :64], u32) + u32(0x8000)
    bo = jax.lax.bitcast_convert_type(p[:, 64:], u32) + u32(0x8000)
    pk = (be & u32(0xFFFF0000)) | (bo >> 16)
    return jax.lax.bitcast_convert_type(pk, jnp.float32)


def _pre_body(h_ref, cp_ref, ws_ref, wd_ref, ts_ref, td_ref):
    hb = h_ref[...]
    cp = cp_ref[...]
    bn = hb.shape[0]
    z = jnp.zeros((bn, 48), jnp.float32)
    ts_ref[...] = jnp.concatenate(
        [_pack_pairs(jnp.dot(hb, ws_ref[...])), cp, z], axis=1)
    td_ref[...] = jnp.concatenate(
        [_pack_pairs(jnp.dot(hb, wd_ref[...])), -cp, z], axis=1)


@functools.lru_cache(maxsize=None)
def _pre_call(n):
    bn = 1000 if n % 1000 == 0 else n
    grid = n // bn
    f32 = jnp.float32
    return pl.pallas_call(
        _pre_body,
        grid=(grid,),
        in_specs=[
            pl.BlockSpec((bn, DIM), lambda i: (i, 0)),
            pl.BlockSpec((bn, 16), lambda i: (i, 0)),
            pl.BlockSpec((DIM, DIM), lambda i: (0, 0)),
            pl.BlockSpec((DIM, DIM), lambda i: (0, 0)),
        ],
        out_specs=[
            pl.BlockSpec((bn, DIM), lambda i: (i, 0)),
            pl.BlockSpec((bn, DIM), lambda i: (i, 0)),
        ],
        out_shape=[
            jax.ShapeDtypeStruct((n, DIM), f32),
            jax.ShapeDtypeStruct((n, DIM), f32),
        ],
    )


def _edge_body(e_real, off, gs_ref, gd_ref, ea_ref, we2_ref, wc1_ref, wrad_ref,
               wea_ref, wc2_ref, be1_ref, be2_ref, bc1_ref, f_ref, s_ref):
    f32 = jnp.float32
    u32 = jnp.uint32
    gs = gs_ref[...]
    gd = gd_ref[...]

    def unpack(x):
        b = jax.lax.bitcast_convert_type(x, u32)
        hi = jax.lax.bitcast_convert_type(b & u32(0xFFFF0000), f32)
        lo = jax.lax.bitcast_convert_type(b << 16, f32)
        return hi, lo

    shi, slo = unpack(gs[:, :64])
    dhi, dlo = unpack(gd[:, :64])
    ssum = jnp.concatenate([shi + dhi, slo + dlo], axis=1)
    cd = gs[:, 64:80] + gd[:, 64:80]
    radial = jnp.sum(cd * cd, axis=1, keepdims=True)
    pre = (ssum + radial * wrad_ref[...] + jnp.dot(ea_ref[...], wea_ref[...])
           + be1_ref[...])
    f1 = pre * jax.nn.sigmoid(pre)
    h2 = jnp.dot(f1, we2_ref[...]) + be2_ref[...]
    f2 = h2 * jax.nn.sigmoid(h2)
    h3 = jnp.dot(f2, wc1_ref[...]) + bc1_ref[...]
    f3 = h3 * jax.nn.sigmoid(h3)
    t = jnp.sum(f3 * wc2_ref[...], axis=1, keepdims=True)
    eid = (off + pl.program_id(0) * BE
           + lax.broadcasted_iota(jnp.int32, (BE, 1), 0))
    valid = eid < e_real
    f_ref[...] = jnp.where(valid, f2, 0.0)
    sm = cd * t
    lane = lax.broadcasted_iota(jnp.int32, (BE, 16), 1)
    sm = jnp.where(lane == 3, 1.0, sm)
    s_ref[...] = jnp.where(valid, sm, 0.0)


@functools.lru_cache(maxsize=None)
def _edge_call(ep, e_real, off):
    f32 = jnp.float32
    full = lambda i: (0, 0)
    return pl.pallas_call(
        functools.partial(_edge_body, e_real, off),
        grid=(ep // BE,),
        in_specs=[
            pl.BlockSpec((BE, DIM), lambda i: (i, 0)),
            pl.BlockSpec((BE, DIM), lambda i: (i, 0)),
            pl.BlockSpec((BE, DE), lambda i: (i, 0)),
            pl.BlockSpec((HID, HID), full),
            pl.BlockSpec((HID, HID), full),
            pl.BlockSpec((1, HID), full),
            pl.BlockSpec((DE, HID), full),
            pl.BlockSpec((1, HID), full),
            pl.BlockSpec((1, HID), full),
            pl.BlockSpec((1, HID), full),
            pl.BlockSpec((1, HID), full),
        ],
        out_specs=[
            pl.BlockSpec((BE, DIM), lambda i: (i, 0)),
            pl.BlockSpec((BE, 16), lambda i: (i, 0)),
        ],
        out_shape=[
            jax.ShapeDtypeStruct((ep, DIM), f32),
            jax.ShapeDtypeStruct((ep, 16), f32),
        ],
    )


def _node_body(h_ref, cp_ref, pf_ref, ps_ref, pf2_ref, ps2_ref, wn1h_ref,
               wn1a_ref, wn2_ref, bn1_ref, bn2_ref, ho_ref, co_ref):
    hb = h_ref[...]
    pf = pf_ref[...]
    pf2 = pf2_ref[...]
    aggf = pf[0] + pf[1] + pf2[0] + pf2[1]
    ps = ps_ref[...]
    ps2 = ps2_ref[...]
    aggs = (ps[0, :, :16] + ps[1, :, :16]
            + ps2[0, :, :16] + ps2[1, :, :16])
    cnt = jnp.maximum(aggs[:, 3:4], 1.0)
    lane = lax.broadcasted_iota(jnp.int32, aggs.shape, 1)
    tr = jnp.where(lane < 3, aggs, 0.0)
    co_ref[...] = cp_ref[...] + tr / cnt
    u = (jnp.dot(hb, wn1h_ref[...]) + jnp.dot(aggf, wn1a_ref[...])
         + bn1_ref[...])
    su = u * jax.nn.sigmoid(u)
    ho_ref[...] = hb + jnp.dot(su, wn2_ref[...]) + bn2_ref[...]


@functools.lru_cache(maxsize=None)
def _node_call(n):
    bn = 1000 if n % 1000 == 0 else n
    grid = n // bn
    f32 = jnp.float32
    full = lambda i: (0, 0)
    return pl.pallas_call(
        _node_body,
        grid=(grid,),
        in_specs=[
            pl.BlockSpec((bn, DIM), lambda i: (i, 0)),
            pl.BlockSpec((bn, 16), lambda i: (i, 0)),
            pl.BlockSpec((2, bn, DIM), lambda i: (0, i, 0)),
            pl.BlockSpec((2, bn, DIM), lambda i: (0, i, 0)),
            pl.BlockSpec((2, bn, DIM), lambda i: (0, i, 0)),
            pl.BlockSpec((2, bn, DIM), lambda i: (0, i, 0)),
            pl.BlockSpec((DIM, HID), full),
            pl.BlockSpec((DIM, HID), full),
            pl.BlockSpec((HID, DIM), full),
            pl.BlockSpec((1, HID), full),
            pl.BlockSpec((1, DIM), full),
        ],
        out_specs=[
            pl.BlockSpec((bn, DIM), lambda i: (i, 0)),
            pl.BlockSpec((bn, 16), lambda i: (i, 0)),
        ],
        out_shape=[
            jax.ShapeDtypeStruct((n, DIM), f32),
            jax.ShapeDtypeStruct((n, 16), f32),
        ],
    )


@functools.lru_cache(maxsize=None)
def _gather_call(ep):
    f32 = jnp.float32
    nb = ep // (NW * EB)
    ng = nb // GRP
    mesh = plsc.VectorSubcoreMesh(core_axis_name="c", subcore_axis_name="s")

    @functools.partial(
        pl.kernel,
        mesh=mesh,
        out_type=[
            jax.ShapeDtypeStruct((ep, DIM), f32),
            jax.ShapeDtypeStruct((ep, DIM), f32),
        ],
        scratch_types=[
            pltpu.VMEM((GRP, EB), jnp.int32),
            pltpu.VMEM((GRP, EB), jnp.int32),
            pltpu.VMEM((EB, DIM), f32),
            pltpu.VMEM((EB, DIM), f32),
            pltpu.VMEM((EB, DIM), f32),
            pltpu.VMEM((EB, DIM), f32),
            pltpu.SemaphoreType.DMA,
            pltpu.SemaphoreType.DMA,
            pltpu.SemaphoreType.DMA,
            pltpu.SemaphoreType.DMA,
        ],
    )
    def gather_k(tsrc, tdst, rows, cols, gs_out, gd_out, rowv, colv,
                 gs0, gs1, gd0, gd1, gsem0, gsem1, wsem0, wsem1):
        wid = lax.axis_index("s") * NC + lax.axis_index("c")
        base_row = wid * nb
        gsb = (gs0, gs1)
        gdb = (gd0, gd1)
        gsem = (gsem0, gsem1)
        wsem = (wsem0, wsem1)

        def body(g, carry):
            r0 = pl.multiple_of(base_row + g * GRP, GRP)
            pltpu.sync_copy(rows.at[pl.ds(r0, GRP)], rowv)
            pltpu.sync_copy(cols.at[pl.ds(r0, GRP)], colv)
            gh = [None, None]
            wh = [None, None]
            gh[0] = (pltpu.async_copy(tsrc.at[rowv.at[0]], gsb[0], gsem[0]),
                     pltpu.async_copy(tdst.at[colv.at[0]], gdb[0], gsem[0]))
            for j in range(GRP):
                b = j & 1
                nx = 1 - b
                for c in gh[b]:
                    c.wait()
                if j < GRP - 1:
                    if wh[nx] is not None:
                        for w in wh[nx]:
                            w.wait()
                    gh[nx] = (
                        pltpu.async_copy(tsrc.at[rowv.at[j + 1]], gsb[nx],
                                         gsem[nx]),
                        pltpu.async_copy(tdst.at[colv.at[j + 1]], gdb[nx],
                                         gsem[nx]))
                e0 = pl.multiple_of((r0 + j) * EB, EB)
                wh[b] = (
                    pltpu.async_copy(gsb[b], gs_out.at[pl.ds(e0, EB)],
                                     wsem[b]),
                    pltpu.async_copy(gdb[b], gd_out.at[pl.ds(e0, EB)],
                                     wsem[b]))
            for pair in wh:
                for w in pair:
                    w.wait()
            return carry

        lax.fori_loop(0, ng, body, 0)

    return gather_k


@functools.lru_cache(maxsize=None)
def _scatter_call(ep, n):
    f32 = jnp.float32
    nb = ep // (NW * EB)
    ng = nb // GRP
    cs = EB
    mesh = plsc.VectorSubcoreMesh(core_axis_name="c", subcore_axis_name="s")

    @functools.partial(
        pl.kernel,
        mesh=mesh,
        out_type=[
            jax.ShapeDtypeStruct((NC, n, DIM), f32),
            jax.ShapeDtypeStruct((NC, n, DIM), f32),
        ],
        scratch_types=[
            pltpu.VMEM((GRP, EB), jnp.int32),
            pltpu.VMEM((cs, DIM), f32),
            pltpu.VMEM((16, DIM), f32),
            pltpu.VMEM((16, DIM), f32),
            pltpu.VMEM((cs, DIM), f32),
            pltpu.VMEM_SHARED((n, DIM), f32),
            pltpu.SemaphoreType.DMA,
            pltpu.SemaphoreType.DMA,
        ],
    )
    def scatter_k(feat, small, rows, pf_out, ps_out, idxv, bf, bs16a, bs16b,
                  bex, af, lsem0, lsem1):
        cid = lax.axis_index("c")
        sid = lax.axis_index("s")
        wid = sid * NC + cid
        stride = (n // (NS * 8)) * 8
        swid = -(-(n - (NS - 1) * stride) // cs) * cs
        base_n = pl.multiple_of(sid * stride, 8)
        base_row = wid * nb

        def zero_buf(buf):
            def zf(i, carry):
                buf[i // 8, pl.ds((i % 8) * LANES, LANES)] = jnp.zeros(
                    (LANES,), f32)
                return carry
            lax.fori_loop(0, cs * (DIM // LANES), zf, 0)

        def zero_stripe():
            for off in range(0, swid, cs):
                b = pl.multiple_of(base_n + off, 8)
                pltpu.sync_copy(bf, af.at[pl.ds(b, cs)])

        def add_pass(payload):
            bufs = (bf, bex)
            sems = (lsem0, lsem1)

            def body(g, carry):
                r0 = pl.multiple_of(base_row + g * GRP, GRP)
                pltpu.sync_copy(rows.at[pl.ds(r0, GRP)], idxv)
                lh = [None, None]
                e0 = pl.multiple_of(r0 * EB, EB)
                lh[0] = pltpu.async_copy(payload.at[pl.ds(e0, cs)], bufs[0],
                                         sems[0])
                for j in range(GRP):
                    b = j & 1
                    nx = 1 - b
                    lh[b].wait()
                    if j < GRP - 1:
                        e1 = pl.multiple_of((r0 + j + 1) * EB, EB)
                        lh[nx] = pltpu.async_copy(
                            payload.at[pl.ds(e1, cs)], bufs[nx], sems[nx])
                    pltpu.sync_copy(bufs[b], af.at[idxv.at[j]], add=True)
                return carry
            lax.fori_loop(0, ng, body, 0)

        def small_pass(spacked):
            bufs = (bs16a, bs16b)
            sems = (lsem0, lsem1)

            def body(g, carry):
                r0 = pl.multiple_of(base_row + g * GRP, GRP)
                pltpu.sync_copy(rows.at[pl.ds(r0, GRP)], idxv)
                lh = [None, None]
                p0 = pl.multiple_of(r0 * 16, 16)
                lh[0] = pltpu.async_copy(spacked.at[pl.ds(p0, 16)], bufs[0],
                                         sems[0])
                for j in range(GRP):
                    b = j & 1
                    nx = 1 - b
                    lh[b].wait()
                    if j < GRP - 1:
                        p1 = pl.multiple_of((r0 + j + 1) * 16, 16)
                        lh[nx] = pltpu.async_copy(
                            spacked.at[pl.ds(p1, 16)], bufs[nx], sems[nx])
                    for r in range(16):
                        for k in range(8):
                            bex[8 * r + k, pl.ds(0, LANES)] = (
                                bufs[b][r, pl.ds(k * LANES, LANES)])
                    pltpu.sync_copy(bex, af.at[idxv.at[j]], add=True)
                return carry
            lax.fori_loop(0, ng, body, 0)

        def drain(out):
            for off in range(0, swid, cs):
                b = pl.multiple_of(base_n + off, 8)
                pltpu.sync_copy(af.at[pl.ds(b, cs)], bf)
                pltpu.sync_copy(bf, out.at[cid, pl.ds(b, cs)])

        zero_buf(bf)
        zero_stripe()
        plsc.subcore_barrier()
        add_pass(feat)
        plsc.subcore_barrier()
        drain(pf_out)
        plsc.subcore_barrier()
        zero_buf(bf)
        zero_buf(bex)
        zero_stripe()
        plsc.subcore_barrier()
        small_pass(small)
        plsc.subcore_barrier()
        drain(ps_out)

    return scatter_k


def kernel(h, edge_index, coord, edge_attr, We1, be1, We2, be2, Wn1, bn1,
           Wn2, bn2, Wc1, bc1, Wc2):
    f32 = jnp.float32
    n = h.shape[0]
    e_real = edge_index.shape[1]
    ep = -(-e_real // ALIGN) * ALIGN
    pad = ep - e_real

    row = edge_index[0].astype(jnp.int32)
    col = edge_index[1].astype(jnp.int32)
    rowp = jnp.concatenate([row, jnp.zeros((pad,), jnp.int32)]).reshape(
        ep // EB, EB)
    colp = jnp.concatenate([col, jnp.zeros((pad,), jnp.int32)]).reshape(
        ep // EB, EB)
    eap = jnp.concatenate([edge_attr.astype(f32),
                           jnp.zeros((pad, DE), f32)], axis=0)
    cp = jnp.concatenate([coord.astype(f32), jnp.zeros((n, 13), f32)], axis=1)

    perm = jnp.arange(HID).reshape(HID // 2, 2).T.reshape(HID)
    Wsrc = We1[:DIM, perm]
    Wdst = We1[DIM:2 * DIM, perm]
    wrad = We1[2 * DIM:2 * DIM + 1, perm]
    Wea = We1[2 * DIM + 1:, perm]
    We2p = We2[perm, :]
    wc2r = Wc2.reshape(1, HID)
    Wn1h = Wn1[:DIM]
    Wn1a = Wn1[DIM:]
    be1r = be1[perm].reshape(1, HID)
    be2r = be2.reshape(1, HID)
    bc1r = bc1.reshape(1, HID)
    bn1r = bn1.reshape(1, HID)
    bn2r = bn2.reshape(1, DIM)

    ts, td = _pre_call(n)(h, cp, Wsrc, Wdst)
    half = ep // 2
    hbr = half // EB
    g1s, g1d = _gather_call(half)(ts, td, rowp[:hbr], colp[:hbr])
    g2s, g2d = _gather_call(half)(ts, td, rowp[hbr:], colp[hbr:])
    f1, s1 = _edge_call(half, e_real, 0)(g1s, g1d, eap[:half], We2p, Wc1,
                                         wrad, Wea, wc2r, be1r, be2r, bc1r)
    f2, s2 = _edge_call(half, e_real, half)(g2s, g2d, eap[half:], We2p, Wc1,
                                            wrad, Wea, wc2r, be1r, be2r, bc1r)
    pf1, ps1 = _scatter_call(half, n)(f1, s1.reshape(half // 8, 128),
                                      rowp[:hbr])
    pf2, ps2 = _scatter_call(half, n)(f2, s2.reshape(half // 8, 128),
                                      rowp[hbr:])
    ho, co = _node_call(n)(h, cp, pf1, ps1, pf2, ps2, Wn1h, Wn1a, Wn2,
                           bn1r, bn2r)
    return (ho, co[:, :3], edge_attr)
